# baseline (speedup 1.0000x reference)
import functools

import jax
import jax.numpy as jnp
from jax import lax
from jax.experimental import pallas as pl
from jax.experimental.pallas import tpu as pltpu
from jax.experimental.pallas import tpu_sc as plsc

N = 10000
E = 320000
H = 128
MAXZ = 1000
NGRAPH = 512

CH = 128
EPC = 80
GCH = 80
NGC = N // GCH
NROW = N + GCH
NRC = NROW // GCH


def _sc_mesh():
    return plsc.VectorSubcoreMesh(core_axis_name="c", subcore_axis_name="s")


def _rows_loop(ns, sid, nblocks, fn):
    def body(i, _):
        j = sid + i * ns

        @pl.when(j < nblocks)
        def _():
            fn(pl.multiple_of(j * GCH, GCH))
        return 0
    lax.fori_loop(0, (nblocks + ns - 1) // ns, body, 0)


@functools.lru_cache(maxsize=None)
def _make_embed_cnt():
    mesh = _sc_mesh()
    nc, ns = mesh.num_cores, mesh.num_subcores
    nw = nc * ns

    @functools.partial(
        pl.kernel,
        out_type=[
            jax.ShapeDtypeStruct((N, H), jnp.float32),
            jax.ShapeDtypeStruct((nc, N, H), jnp.float32),
        ],
        mesh=mesh,
        scratch_types=[
            pltpu.VMEM((GCH,), jnp.int32),
            pltpu.VMEM((GCH, H), jnp.float32),
            pltpu.VMEM((CH,), jnp.int32),
            pltpu.VMEM((CH,), jnp.int32),
            pltpu.VMEM((CH, H), jnp.float32),
            pltpu.VMEM_SHARED((NROW, H), jnp.float32),
            pltpu.SemaphoreType.DMA,
            pltpu.SemaphoreType.DMA,
            pltpu.SemaphoreType.DMA,
        ],
    )
    def k(z_hbm, ztab_hbm, dst_hbm, zeros_hbm, ones_hbm, h0_hbm, cntp_hbm,
          zidx_v, grows_v, didx0_v, didx1_v, ones_v, acc, sem, semd0, semd1):
        cid = lax.axis_index("c")
        sid = lax.axis_index("s")
        wid = sid * nc + cid

        pltpu.sync_copy(ones_hbm, ones_v)
        _rows_loop(ns, sid, NRC, lambda base: pltpu.sync_copy(
            zeros_hbm.at[pl.ds(base, GCH)], acc.at[pl.ds(base, GCH)]))
        plsc.subcore_barrier()

        def gbody(i, _):
            kk = wid + i * nw

            @pl.when(kk < NGC)
            def _():
                base = pl.multiple_of(kk * GCH, GCH)
                pltpu.sync_copy(z_hbm.at[pl.ds(base, GCH)], zidx_v)
                pltpu.async_copy(ztab_hbm.at[zidx_v], grows_v, sem).wait()
                pltpu.sync_copy(grows_v, h0_hbm.at[pl.ds(base, GCH)])
            return 0
        lax.fori_loop(0, (NGC + nw - 1) // nw, gbody, 0)

        def ebase(j):
            return pl.multiple_of((wid * EPC + j) * CH, CH)

        pltpu.async_copy(dst_hbm.at[pl.ds(ebase(0), CH)], didx0_v, semd0)

        def cbody(jo, _):
            j0 = jo * 2
            pltpu.async_copy(dst_hbm.at[pl.ds(ebase(j0 + 1), CH)], didx1_v,
                             semd1)
            pltpu.make_async_copy(dst_hbm.at[pl.ds(ebase(j0), CH)], didx0_v,
                                  semd0).wait()
            pltpu.sync_copy(ones_v, acc.at[didx0_v], add=True)

            @pl.when(jo + 1 < EPC // 2)
            def _():
                pltpu.async_copy(dst_hbm.at[pl.ds(ebase(j0 + 2), CH)],
                                 didx0_v, semd0)
            pltpu.make_async_copy(dst_hbm.at[pl.ds(ebase(j0 + 1), CH)],
                                  didx1_v, semd1).wait()
            pltpu.sync_copy(ones_v, acc.at[didx1_v], add=True)
            return 0
        lax.fori_loop(0, EPC // 2, cbody, 0)

        plsc.subcore_barrier()
        _rows_loop(ns, sid, NGC, lambda base: pltpu.sync_copy(
            acc.at[pl.ds(base, GCH)], cntp_hbm.at[cid, pl.ds(base, GCH)]))

    return k


@functools.lru_cache(maxsize=None)
def _make_agg():
    mesh = _sc_mesh()
    nc, ns = mesh.num_cores, mesh.num_subcores

    @functools.partial(
        pl.kernel,
        out_type=jax.ShapeDtypeStruct((nc, N, H), jnp.float32),
        mesh=mesh,
        scratch_types=[
            pltpu.VMEM((EPC * CH,), jnp.int32),
            pltpu.VMEM((CH,), jnp.int32),
            pltpu.VMEM((CH,), jnp.int32),
            pltpu.VMEM((CH, H), jnp.float32),
            pltpu.VMEM((CH, H), jnp.float32),
            pltpu.VMEM_SHARED((NROW, H), jnp.float32),
            pltpu.SemaphoreType.DMA,
            pltpu.SemaphoreType.DMA,
            pltpu.SemaphoreType.DMA,
            pltpu.SemaphoreType.DMA,
        ],
    )
    def k(h_hbm, src_hbm, dst_hbm, zeros_hbm, part_hbm,
          sidxf_v, didx0_v, didx1_v, rows0_v, rows1_v,
          acc, sem0, sem1, semd0, semd1):
        cid = lax.axis_index("c")
        sid = lax.axis_index("s")
        wid = sid * nc + cid

        _rows_loop(ns, sid, NRC, lambda base: pltpu.sync_copy(
            zeros_hbm.at[pl.ds(base, GCH)], acc.at[pl.ds(base, GCH)]))

        def ebase(j):
            return pl.multiple_of((wid * EPC + j) * CH, CH)

        pltpu.sync_copy(src_hbm.at[pl.ds(ebase(0), EPC * CH)], sidxf_v)
        plsc.subcore_barrier()

        def sslice(j):
            return sidxf_v.at[pl.ds(j * CH, CH)]

        pltpu.async_copy(dst_hbm.at[pl.ds(ebase(0), CH)], didx0_v, semd0)
        pltpu.async_copy(h_hbm.at[sslice(0)], rows0_v, sem0)

        def body(jo, _):
            j0 = jo * 2
            pltpu.async_copy(dst_hbm.at[pl.ds(ebase(j0 + 1), CH)], didx1_v,
                             semd1)
            pltpu.async_copy(h_hbm.at[sslice(j0 + 1)], rows1_v, sem1)
            pltpu.make_async_copy(h_hbm.at[sslice(j0)], rows0_v, sem0).wait()
            pltpu.make_async_copy(dst_hbm.at[pl.ds(ebase(j0), CH)], didx0_v,
                                  semd0).wait()
            pltpu.sync_copy(rows0_v, acc.at[didx0_v], add=True)

            @pl.when(jo + 1 < EPC // 2)
            def _():
                pltpu.async_copy(dst_hbm.at[pl.ds(ebase(j0 + 2), CH)],
                                 didx0_v, semd0)
                pltpu.async_copy(h_hbm.at[sslice(j0 + 2)], rows0_v, sem0)
            pltpu.make_async_copy(h_hbm.at[sslice(j0 + 1)], rows1_v,
                                  sem1).wait()
            pltpu.make_async_copy(dst_hbm.at[pl.ds(ebase(j0 + 1), CH)],
                                  didx1_v, semd1).wait()
            pltpu.sync_copy(rows1_v, acc.at[didx1_v], add=True)
            return 0
        lax.fori_loop(0, EPC // 2, body, 0)

        plsc.subcore_barrier()
        _rows_loop(ns, sid, NGC, lambda base: pltpu.sync_copy(
            acc.at[pl.ds(base, GCH)], part_hbm.at[cid, pl.ds(base, GCH)]))

    return k


def _dot_t(a, w):
    return lax.dot_general(a, w, (((1,), (1,)), ((), ())),
                           preferred_element_type=jnp.float32)


_TCB = 1000


def _tc_layer0(part, cntp, h, wl, wr, b):
    def body(part_ref, cntp_ref, h_ref, wl_ref, wr_ref, b_ref,
             h1_ref, inv_ref):
        cnt = jnp.maximum(cntp_ref[0][:, 0:1] + cntp_ref[1][:, 0:1], 1.0)
        inv_ref[...] = cnt
        mean = (part_ref[0] + part_ref[1]) / cnt
        out = _dot_t(mean, wl_ref[...]) + _dot_t(h_ref[...], wr_ref[...])
        out = out + b_ref[...][None, :]
        h1_ref[...] = jnp.maximum(out, 0.0)

    nb = N // _TCB
    return pl.pallas_call(
        body,
        grid=(nb,),
        in_specs=[
            pl.BlockSpec((2, _TCB, H), lambda i: (0, i, 0)),
            pl.BlockSpec((2, _TCB, H), lambda i: (0, i, 0)),
            pl.BlockSpec((_TCB, H), lambda i: (i, 0)),
            pl.BlockSpec((H, H), lambda i: (0, 0)),
            pl.BlockSpec((H, H), lambda i: (0, 0)),
            pl.BlockSpec((H,), lambda i: (0,)),
        ],
        out_specs=[
            pl.BlockSpec((_TCB, H), lambda i: (i, 0)),
            pl.BlockSpec((_TCB, 1), lambda i: (i, 0)),
        ],
        out_shape=[
            jax.ShapeDtypeStruct((N, H), jnp.float32),
            jax.ShapeDtypeStruct((N, 1), jnp.float32),
        ],
    )(part, cntp, h, wl, wr, b)


def _tc_layer(part, inv, h, wl, wr, b):
    def body(part_ref, inv_ref, h_ref, wl_ref, wr_ref, b_ref, h1_ref):
        mean = (part_ref[0] + part_ref[1]) / inv_ref[...]
        out = _dot_t(mean, wl_ref[...]) + _dot_t(h_ref[...], wr_ref[...])
        out = out + b_ref[...][None, :]
        h1_ref[...] = jnp.maximum(out, 0.0)

    nb = N // _TCB
    return pl.pallas_call(
        body,
        grid=(nb,),
        in_specs=[
            pl.BlockSpec((2, _TCB, H), lambda i: (0, i, 0)),
            pl.BlockSpec((_TCB, 1), lambda i: (i, 0)),
            pl.BlockSpec((_TCB, H), lambda i: (i, 0)),
            pl.BlockSpec((H, H), lambda i: (0, 0)),
            pl.BlockSpec((H, H), lambda i: (0, 0)),
            pl.BlockSpec((H,), lambda i: (0,)),
        ],
        out_specs=pl.BlockSpec((_TCB, H), lambda i: (i, 0)),
        out_shape=jax.ShapeDtypeStruct((N, H), jnp.float32),
    )(part, inv, h, wl, wr, b)


def _tc_final(part, inv, h, batch, wl, wr, b, w1, b1l, w2, b2l):
    nchunks = 10
    rows_per = N // nchunks

    def body(part_ref, inv_ref, h_ref, batch_ref, wl_ref, wr_ref, b_ref,
             w1_ref, b1l_ref, w2_ref, out_ref):
        mean = (part_ref[0] + part_ref[1]) / inv_ref[...]
        h3 = _dot_t(mean, wl_ref[...]) + _dot_t(h_ref[...], wr_ref[...])
        h3 = h3 + b_ref[...][None, :]

        gi = lax.broadcasted_iota(jnp.int32, (1, NGRAPH), 1)
        ci = jnp.zeros((1, NGRAPH), jnp.int32)
        for t in range(nchunks):
            bc = batch_ref[pl.ds(t * rows_per, rows_per)]
            ci = ci + jnp.sum((bc[:, None] < gi).astype(jnp.int32),
                              axis=0, keepdims=True)
        ci_a = jnp.minimum(ci, N - 1).reshape(NGRAPH, 1)
        ci_b = jnp.minimum(ci + 1, N - 1).reshape(NGRAPH, 1)

        pa = jnp.zeros((NGRAPH, H), jnp.float32)
        pb = jnp.zeros((NGRAPH, H), jnp.float32)
        for t in range(nchunks):
            rows = h3[t * rows_per:(t + 1) * rows_per]
            nid = (lax.broadcasted_iota(jnp.int32, (NGRAPH, rows_per), 1)
                   + t * rows_per)
            oh_a = (ci_a == nid).astype(jnp.float32)
            oh_b = (ci_b == nid).astype(jnp.float32)
            pa = pa + jnp.dot(oh_a, rows, preferred_element_type=jnp.float32)
            pb = pb + jnp.dot(oh_b, rows, preferred_element_type=jnp.float32)

        p = pa * pb
        q = jnp.maximum(_dot_t(p, w1_ref[...]) + b1l_ref[...][None, :], 0.0)
        out_ref[...] = _dot_t(q, w2_ref[...])

    out = pl.pallas_call(
        body,
        out_shape=jax.ShapeDtypeStruct((NGRAPH, 1), jnp.float32),
    )(part, inv, h, batch, wl, wr, b, w1, b1l, w2)
    return out + b2l[None, :]


def kernel(z, edge_index, batch, x, edge_weight, node_id, z_table,
           Wl0, Wr0, b0, Wl1, Wr1, b1, Wl2, Wr2, b2, W1, b1l, W2, b2l):
    src = edge_index[0].astype(jnp.int32)
    dst = edge_index[1].astype(jnp.int32)
    z = z.astype(jnp.int32)
    batch = batch.astype(jnp.int32)

    mesh = _sc_mesh()
    nw = mesh.num_cores * mesh.num_subcores
    epad = nw * EPC * CH - E
    pad_i = jnp.arange(epad, dtype=jnp.int32)
    srcp = jnp.concatenate([src, pad_i % N])
    dstp = jnp.concatenate([dst, N + (pad_i % GCH)])

    zeros_nh = jnp.zeros((NROW, H), jnp.float32)
    ones_ch = jnp.ones((CH, H), jnp.float32)

    h0, cntp = _make_embed_cnt()(z, z_table, dstp, zeros_nh, ones_ch)
    part0 = _make_agg()(h0, srcp, dstp, zeros_nh)
    h1, inv = _tc_layer0(part0, cntp, h0, Wl0, Wr0, b0)
    part1 = _make_agg()(h1, srcp, dstp, zeros_nh)
    h2 = _tc_layer(part1, inv, h1, Wl1, Wr1, b1)
    part2 = _make_agg()(h2, srcp, dstp, zeros_nh)
    return _tc_final(part2, inv, h2, batch, Wl2, Wr2, b2, W1, b1l, W2, b2l)


# speedup vs baseline: 1.0723x; 1.0723x over previous
import functools

import jax
import jax.numpy as jnp
from jax import lax
from jax.experimental import pallas as pl
from jax.experimental.pallas import tpu as pltpu
from jax.experimental.pallas import tpu_sc as plsc

N = 10000
E = 320000
H = 128
MAXZ = 1000
NGRAPH = 512

CH = 128
EPC = 80
GCH = 80
NGC = N // GCH
NROW = N + GCH
NRC = NROW // GCH


def _sc_mesh():
    return plsc.VectorSubcoreMesh(core_axis_name="c", subcore_axis_name="s")


def _rows_loop(ns, sid, nblocks, fn):
    def body(i, _):
        j = sid + i * ns

        @pl.when(j < nblocks)
        def _():
            fn(pl.multiple_of(j * GCH, GCH))
        return 0
    lax.fori_loop(0, (nblocks + ns - 1) // ns, body, 0)


@functools.lru_cache(maxsize=None)
def _make_embed():
    mesh = _sc_mesh()
    nc, ns = mesh.num_cores, mesh.num_subcores
    nw = nc * ns

    @functools.partial(
        pl.kernel,
        out_type=jax.ShapeDtypeStruct((N, H), jnp.float32),
        mesh=mesh,
        scratch_types=[
            pltpu.VMEM((GCH,), jnp.int32),
            pltpu.VMEM((GCH, H), jnp.float32),
            pltpu.SemaphoreType.DMA,
        ],
    )
    def k(z_hbm, ztab_hbm, h0_hbm, zidx_v, grows_v, sem):
        cid = lax.axis_index("c")
        sid = lax.axis_index("s")
        wid = sid * nc + cid

        def gbody(i, _):
            kk = wid + i * nw

            @pl.when(kk < NGC)
            def _():
                base = pl.multiple_of(kk * GCH, GCH)
                pltpu.sync_copy(z_hbm.at[pl.ds(base, GCH)], zidx_v)
                pltpu.async_copy(ztab_hbm.at[zidx_v], grows_v, sem).wait()
                pltpu.sync_copy(grows_v, h0_hbm.at[pl.ds(base, GCH)])
            return 0
        lax.fori_loop(0, (NGC + nw - 1) // nw, gbody, 0)

    return k


@functools.lru_cache(maxsize=None)
def _make_agg():
    mesh = _sc_mesh()
    nc, ns = mesh.num_cores, mesh.num_subcores

    @functools.partial(
        pl.kernel,
        out_type=jax.ShapeDtypeStruct((nc, N, H), jnp.float32),
        mesh=mesh,
        scratch_types=[
            pltpu.VMEM((EPC * CH,), jnp.int32),
            pltpu.VMEM((CH,), jnp.int32),
            pltpu.VMEM((CH,), jnp.int32),
            pltpu.VMEM((CH, H), jnp.float32),
            pltpu.VMEM((CH, H), jnp.float32),
            pltpu.VMEM_SHARED((NROW, H), jnp.float32),
            pltpu.SemaphoreType.DMA,
            pltpu.SemaphoreType.DMA,
            pltpu.SemaphoreType.DMA,
            pltpu.SemaphoreType.DMA,
        ],
    )
    def k(h_hbm, src_hbm, dst_hbm, zeros_hbm, part_hbm,
          sidxf_v, didx0_v, didx1_v, rows0_v, rows1_v,
          acc, sem0, sem1, semd0, semd1):
        cid = lax.axis_index("c")
        sid = lax.axis_index("s")
        wid = sid * nc + cid

        _rows_loop(ns, sid, NRC, lambda base: pltpu.sync_copy(
            zeros_hbm.at[pl.ds(base, GCH)], acc.at[pl.ds(base, GCH)]))

        def ebase(j):
            return pl.multiple_of((wid * EPC + j) * CH, CH)

        pltpu.sync_copy(src_hbm.at[pl.ds(ebase(0), EPC * CH)], sidxf_v)
        plsc.subcore_barrier()

        def sslice(j):
            return sidxf_v.at[pl.ds(j * CH, CH)]

        pltpu.async_copy(dst_hbm.at[pl.ds(ebase(0), CH)], didx0_v, semd0)
        pltpu.async_copy(h_hbm.at[sslice(0)], rows0_v, sem0)

        def body(jo, _):
            j0 = jo * 2
            pltpu.async_copy(dst_hbm.at[pl.ds(ebase(j0 + 1), CH)], didx1_v,
                             semd1)
            pltpu.async_copy(h_hbm.at[sslice(j0 + 1)], rows1_v, sem1)
            pltpu.make_async_copy(h_hbm.at[sslice(j0)], rows0_v, sem0).wait()
            pltpu.make_async_copy(dst_hbm.at[pl.ds(ebase(j0), CH)], didx0_v,
                                  semd0).wait()
            pltpu.sync_copy(rows0_v, acc.at[didx0_v], add=True)

            @pl.when(jo + 1 < EPC // 2)
            def _():
                pltpu.async_copy(dst_hbm.at[pl.ds(ebase(j0 + 2), CH)],
                                 didx0_v, semd0)
                pltpu.async_copy(h_hbm.at[sslice(j0 + 2)], rows0_v, sem0)
            pltpu.make_async_copy(h_hbm.at[sslice(j0 + 1)], rows1_v,
                                  sem1).wait()
            pltpu.make_async_copy(dst_hbm.at[pl.ds(ebase(j0 + 1), CH)],
                                  didx1_v, semd1).wait()
            pltpu.sync_copy(rows1_v, acc.at[didx1_v], add=True)
            return 0
        lax.fori_loop(0, EPC // 2, body, 0)

        plsc.subcore_barrier()
        _rows_loop(ns, sid, NGC, lambda base: pltpu.sync_copy(
            acc.at[pl.ds(base, GCH)], part_hbm.at[cid, pl.ds(base, GCH)]))

    return k


NQ = 79
NF = NQ * H


def _tc_pre(dst, batch):
    ec = 1280
    nst = E // ec
    nbc = 10
    rows_per = N // nbc

    def body(dst_ref, batch_ref, cnt_ref, ci_ref, m_ref):
        i = pl.program_id(0)

        @pl.when(i == 0)
        def _():
            m_ref[...] = jnp.zeros((NQ, H), jnp.float32)
            gi = lax.broadcasted_iota(jnp.int32, (1, NGRAPH), 1)
            ci = jnp.zeros((1, NGRAPH), jnp.int32)
            for t in range(nbc):
                bc = batch_ref[pl.ds(t * rows_per, rows_per)]
                ci = ci + jnp.sum((bc[:, None] < gi).astype(jnp.int32),
                                  axis=0, keepdims=True)
            ci_t = ci.reshape(NGRAPH, 1)
            ci_ref[...] = jnp.concatenate(
                [jnp.minimum(ci_t, N - 1), jnp.minimum(ci_t + 1, N - 1)],
                axis=1)

        d = dst_ref[0, 0]
        ohq = ((d[:, None] >> 7) ==
               lax.broadcasted_iota(jnp.int32, (ec, NQ), 1)
               ).astype(jnp.float32)
        ohr = ((d[:, None] & 127) ==
               lax.broadcasted_iota(jnp.int32, (ec, H), 1)
               ).astype(jnp.float32)
        m_ref[...] = m_ref[...] + lax.dot_general(
            ohq, ohr, (((0,), (0,)), ((), ())),
            preferred_element_type=jnp.float32)

        @pl.when(i == nst - 1)
        def _():
            cnt_ref[...] = m_ref[...]

    return pl.pallas_call(
        body,
        grid=(nst,),
        in_specs=[
            pl.BlockSpec((1, 1, ec), lambda i: (i, 0, 0)),
            pl.BlockSpec((N,), lambda i: (0,)),
        ],
        out_specs=[
            pl.BlockSpec((NQ, H), lambda i: (0, 0)),
            pl.BlockSpec((NGRAPH, 2), lambda i: (0, 0)),
        ],
        out_shape=[
            jax.ShapeDtypeStruct((NQ, H), jnp.float32),
            jax.ShapeDtypeStruct((NGRAPH, 2), jnp.int32),
        ],
        scratch_shapes=[pltpu.VMEM((NQ, H), jnp.float32)],
    )(dst.reshape(nst, 1, ec), batch)


def _dot_t(a, w):
    return lax.dot_general(a, w, (((1,), (1,)), ((), ())),
                           preferred_element_type=jnp.float32)


def _tc_layer0(part, cntf, h, wl, wr, b):
    def body(part_ref, cntf_ref, h_ref, wl_ref, wr_ref, b_ref,
             h1_ref, inv_ref):
        cnt = jnp.maximum(cntf_ref[pl.ds(0, N)], 1.0)
        inv_ref[...] = cnt
        mean = (part_ref[0] + part_ref[1]) / cnt
        out = _dot_t(mean, wl_ref[...]) + _dot_t(h_ref[...], wr_ref[...])
        out = out + b_ref[...][None, :]
        h1_ref[...] = jnp.maximum(out, 0.0)

    return pl.pallas_call(
        body,
        out_shape=[
            jax.ShapeDtypeStruct((N, H), jnp.float32),
            jax.ShapeDtypeStruct((N, 1), jnp.float32),
        ],
    )(part, cntf, h, wl, wr, b)


def _tc_layer(part, inv, h, wl, wr, b):
    def body(part_ref, inv_ref, h_ref, wl_ref, wr_ref, b_ref, h1_ref):
        mean = (part_ref[0] + part_ref[1]) / inv_ref[...]
        out = _dot_t(mean, wl_ref[...]) + _dot_t(h_ref[...], wr_ref[...])
        out = out + b_ref[...][None, :]
        h1_ref[...] = jnp.maximum(out, 0.0)

    return pl.pallas_call(
        body,
        out_shape=jax.ShapeDtypeStruct((N, H), jnp.float32),
    )(part, inv, h, wl, wr, b)


def _tc_final(part, inv, h, ci, wl, wr, b, w1, b1l, w2, b2l):
    nchunks = 10
    rows_per = N // nchunks

    def body(part_ref, inv_ref, h_ref, ci_ref, wl_ref, wr_ref, b_ref,
             w1_ref, b1l_ref, w2_ref, out_ref):
        mean = (part_ref[0] + part_ref[1]) / inv_ref[...]
        h3 = _dot_t(mean, wl_ref[...]) + _dot_t(h_ref[...], wr_ref[...])
        h3 = h3 + b_ref[...][None, :]

        ci_a = ci_ref[...][:, 0:1]
        ci_b = ci_ref[...][:, 1:2]

        pa = jnp.zeros((NGRAPH, H), jnp.float32)
        pb = jnp.zeros((NGRAPH, H), jnp.float32)
        for t in range(nchunks):
            rows = h3[t * rows_per:(t + 1) * rows_per]
            nid = (lax.broadcasted_iota(jnp.int32, (NGRAPH, rows_per), 1)
                   + t * rows_per)
            oh_a = (ci_a == nid).astype(jnp.float32)
            oh_b = (ci_b == nid).astype(jnp.float32)
            pa = pa + jnp.dot(oh_a, rows, preferred_element_type=jnp.float32)
            pb = pb + jnp.dot(oh_b, rows, preferred_element_type=jnp.float32)

        p = pa * pb
        q = jnp.maximum(_dot_t(p, w1_ref[...]) + b1l_ref[...][None, :], 0.0)
        out_ref[...] = _dot_t(q, w2_ref[...])

    out = pl.pallas_call(
        body,
        out_shape=jax.ShapeDtypeStruct((NGRAPH, 1), jnp.float32),
    )(part, inv, h, ci, wl, wr, b, w1, b1l, w2)
    return out + b2l[None, :]


def kernel(z, edge_index, batch, x, edge_weight, node_id, z_table,
           Wl0, Wr0, b0, Wl1, Wr1, b1, Wl2, Wr2, b2, W1, b1l, W2, b2l):
    src = edge_index[0].astype(jnp.int32)
    dst = edge_index[1].astype(jnp.int32)
    z = z.astype(jnp.int32)
    batch = batch.astype(jnp.int32)

    mesh = _sc_mesh()
    nw = mesh.num_cores * mesh.num_subcores
    epad = nw * EPC * CH - E
    pad_i = jnp.arange(epad, dtype=jnp.int32)
    srcp = jnp.concatenate([src, pad_i % N])
    dstp = jnp.concatenate([dst, N + (pad_i % GCH)])

    zeros_nh = jnp.zeros((NROW, H), jnp.float32)

    h0 = _make_embed()(z, z_table)
    cnt2d, ci = _tc_pre(dst, batch)
    cntf = cnt2d.reshape(NF, 1)
    part0 = _make_agg()(h0, srcp, dstp, zeros_nh)
    h1, inv = _tc_layer0(part0, cntf, h0, Wl0, Wr0, b0)
    part1 = _make_agg()(h1, srcp, dstp, zeros_nh)
    h2 = _tc_layer(part1, inv, h1, Wl1, Wr1, b1)
    part2 = _make_agg()(h2, srcp, dstp, zeros_nh)
    return _tc_final(part2, inv, h2, ci, Wl2, Wr2, b2, W1, b1l, W2, b2l)
